# baseline (speedup 1.0000x reference)
import functools

import jax
import jax.numpy as jnp
from jax import lax
from jax.experimental import pallas as pl
from jax.experimental.pallas import tpu as pltpu, tpu_sc as plsc

B = 16384
D = 64
NF = 6
VOCABS = (50, 100, 500, 12, 31, 7)
NC = 2
NS = 16
NW = NC * NS
BPW = B // NW
RND = 8
RPR = BPW // RND
GPR = RPR // 8

_mesh = plsc.VectorSubcoreMesh(core_axis_name="c", subcore_axis_name="s")


@functools.partial(
    pl.kernel,
    out_type=jax.ShapeDtypeStruct((B // 8, 3, 8, 128), jnp.float32),
    mesh=_mesh,
    compiler_params=pltpu.CompilerParams(use_tc_tiling_on_sc=False),
    scratch_types=[
        pltpu.VMEM((NF, BPW), jnp.int32),
        [[pltpu.VMEM((RPR, D), jnp.float32) for _ in range(NF)]
         for _ in range(2)],
        [pltpu.MemorySpace.VMEM_SHARED((v, D), jnp.float32) for v in VOCABS],
        pltpu.SemaphoreType.DMA,
        pltpu.SemaphoreType.DMA,
        pltpu.SemaphoreType.DMA,
        pltpu.SemaphoreType.DMA,
    ],
)
def _emb6(c_i, s_i, p_i, m_i, d_i, w_i,
          c_t, s_t, p_t, m_t, d_t, w_t,
          out_hbm, idx_v, rings, sp_tabs,
          sem_i, sem_g, sem_w0, sem_w1):
    cid = lax.axis_index("c")
    sid = lax.axis_index("s")
    wid = sid * NC + cid
    base = wid * BPW
    sem_w = (sem_w0, sem_w1)
    idxs = (c_i, s_i, p_i, m_i, d_i, w_i)
    tabs = (c_t, s_t, p_t, m_t, d_t, w_t)

    for f in range(NF):
        pltpu.async_copy(idxs[f].at[pl.ds(base, BPW)], idx_v.at[f], sem_i)

    for f in range(NF):
        @pl.when(sid == f)
        def _():
            pltpu.sync_copy(tabs[f], sp_tabs[f])
    plsc.subcore_barrier()

    for f in range(NF):
        pltpu.make_async_copy(idxs[f].at[pl.ds(base, BPW)], idx_v.at[f],
                              sem_i).wait()

    def _fire_writes(r, bufs, sem):
        rb = (base + r * RPR) // 8

        def _wbody(g, carry):
            for f in range(NF):
                pltpu.async_copy(
                    bufs[f].at[pl.ds(g * 8, 8)],
                    out_hbm.at[rb + g, f // 2, :, pl.ds((f % 2) * D, D)],
                    sem)
            return carry

        lax.fori_loop(0, GPR, _wbody, 0)

    def _drain_writes(r, bufs, sem):
        rb = (base + r * RPR) // 8

        def _wbody(g, carry):
            for f in range(NF):
                pltpu.make_async_copy(
                    bufs[f].at[pl.ds(g * 8, 8)],
                    out_hbm.at[rb + g, f // 2, :, pl.ds((f % 2) * D, D)],
                    sem).wait()
            return carry

        lax.fori_loop(0, GPR, _wbody, 0)

    for r in range(RND):
        b = r % 2
        bufs = rings[b]
        if r >= 2:
            _drain_writes(r - 2, bufs, sem_w[b])
        for f in range(NF):
            pltpu.async_copy(
                sp_tabs[f].at[idx_v.at[f, pl.ds(r * RPR, RPR)]],
                bufs[f], sem_g)
        for f in range(NF):
            pltpu.make_async_copy(
                sp_tabs[f].at[idx_v.at[f, pl.ds(r * RPR, RPR)]],
                bufs[f], sem_g).wait()
        _fire_writes(r, bufs, sem_w[b])
    for r in (RND - 2, RND - 1):
        _drain_writes(r, rings[r % 2], sem_w[r % 2])


def kernel(past_num_sold, country, store, product, month, day, dayofweek,
           W_country, W_store, W_product, W_month, W_day, W_dayofweek):
    x4 = _emb6(country, store, product, month, day, dayofweek,
               W_country, W_store, W_product, W_month, W_day, W_dayofweek)
    x_cats = x4.transpose(0, 2, 1, 3).reshape(B, NF * D)
    return (past_num_sold, x_cats)


# speedup vs baseline: 1.0105x; 1.0105x over previous
import functools

import jax
import jax.numpy as jnp
from jax import lax
from jax.experimental import pallas as pl
from jax.experimental.pallas import tpu as pltpu, tpu_sc as plsc

B = 16384
D = 64
NF = 6
VOCABS = (50, 100, 500, 12, 31, 7)
NC = 2
NS = 16
NW = NC * NS
BPW = B // NW
RND = 4
RPR = BPW // RND
GPR = RPR // 8

_mesh = plsc.VectorSubcoreMesh(core_axis_name="c", subcore_axis_name="s")


@functools.partial(
    pl.kernel,
    out_type=jax.ShapeDtypeStruct((B // 8, 3, 8, 128), jnp.float32),
    mesh=_mesh,
    compiler_params=pltpu.CompilerParams(use_tc_tiling_on_sc=False),
    scratch_types=[
        pltpu.VMEM((NF, BPW), jnp.int32),
        [[pltpu.VMEM((RPR, D), jnp.float32) for _ in range(NF)]
         for _ in range(2)],
        [pltpu.MemorySpace.VMEM_SHARED((v, D), jnp.float32) for v in VOCABS],
        pltpu.SemaphoreType.DMA,
        pltpu.SemaphoreType.DMA,
        pltpu.SemaphoreType.DMA,
        pltpu.SemaphoreType.DMA,
    ],
)
def _emb6(c_i, s_i, p_i, m_i, d_i, w_i,
          c_t, s_t, p_t, m_t, d_t, w_t,
          out_hbm, idx_v, rings, sp_tabs,
          sem_i, sem_g, sem_w0, sem_w1):
    cid = lax.axis_index("c")
    sid = lax.axis_index("s")
    wid = sid * NC + cid
    base = wid * BPW
    sem_w = (sem_w0, sem_w1)
    idxs = (c_i, s_i, p_i, m_i, d_i, w_i)
    tabs = (c_t, s_t, p_t, m_t, d_t, w_t)

    for f in range(NF):
        pltpu.async_copy(idxs[f].at[pl.ds(base, BPW)], idx_v.at[f], sem_i)

    for f in range(NF):
        @pl.when(sid == f)
        def _():
            pltpu.sync_copy(tabs[f], sp_tabs[f])
    plsc.subcore_barrier()

    for f in range(NF):
        pltpu.make_async_copy(idxs[f].at[pl.ds(base, BPW)], idx_v.at[f],
                              sem_i).wait()

    def _fire_writes(r, bufs, sem):
        rb = (base + r * RPR) // 8

        def _wbody(g, carry):
            for f in range(NF):
                pltpu.async_copy(
                    bufs[f].at[pl.ds(g * 8, 8)],
                    out_hbm.at[rb + g, f // 2, :, pl.ds((f % 2) * D, D)],
                    sem)
            return carry

        lax.fori_loop(0, GPR, _wbody, 0)

    def _drain_writes(r, bufs, sem):
        rb = (base + r * RPR) // 8

        def _wbody(g, carry):
            for f in range(NF):
                pltpu.make_async_copy(
                    bufs[f].at[pl.ds(g * 8, 8)],
                    out_hbm.at[rb + g, f // 2, :, pl.ds((f % 2) * D, D)],
                    sem).wait()
            return carry

        lax.fori_loop(0, GPR, _wbody, 0)

    for r in range(RND):
        b = r % 2
        bufs = rings[b]
        if r >= 2:
            _drain_writes(r - 2, bufs, sem_w[b])
        for f in range(NF):
            pltpu.async_copy(
                sp_tabs[f].at[idx_v.at[f, pl.ds(r * RPR, RPR)]],
                bufs[f], sem_g)
        for f in range(NF):
            pltpu.make_async_copy(
                sp_tabs[f].at[idx_v.at[f, pl.ds(r * RPR, RPR)]],
                bufs[f], sem_g).wait()
        _fire_writes(r, bufs, sem_w[b])
    for r in (RND - 2, RND - 1):
        _drain_writes(r, rings[r % 2], sem_w[r % 2])


def kernel(past_num_sold, country, store, product, month, day, dayofweek,
           W_country, W_store, W_product, W_month, W_day, W_dayofweek):
    x4 = _emb6(country, store, product, month, day, dayofweek,
               W_country, W_store, W_product, W_month, W_day, W_dayofweek)
    x_cats = x4.transpose(0, 2, 1, 3).reshape(B, NF * D)
    return (past_num_sold, x_cats)
